# Initial kernel scaffold; baseline (speedup 1.0000x reference)
import jax
import jax.numpy as jnp
from jax.experimental import pallas as pl


def kernel(x, resids_positional_encoded, W):
    raise NotImplementedError("write your pallas kernel here")


# speedup vs baseline: 2.2789x; 2.2789x over previous
import functools

import jax
import jax.numpy as jnp
from jax import lax
from jax.experimental import pallas as pl
from jax.experimental.pallas import tpu as pltpu
from jax.experimental.pallas import tpu_sc as plsc

NUM_EMB = 1000
D = 64
N = 819200

NC = 2
NS = 16
NW = NC * NS
LANES = 16

TOKENS_PER_WORKER = N // NW
CHUNK = 128
STEPS = TOKENS_PER_WORKER // CHUNK
SLICES_PER_ROW = D // LANES


def _body(x_hbm, idx_hbm, w_hbm, out_hbm, w_sh, idx_v, acc_v, rows_v, sem):
    cid = lax.axis_index("c")
    sid = lax.axis_index("s")
    wid = sid * NC + cid

    @pl.when(sid == 0)
    def _():
        pltpu.sync_copy(w_hbm, w_sh)

    plsc.subcore_barrier()

    base0 = wid * TOKENS_PER_WORKER

    def step(k, carry):
        base = base0 + k * CHUNK
        pltpu.sync_copy(idx_hbm.at[pl.ds(base, CHUNK)], idx_v)
        pltpu.sync_copy(x_hbm.at[pl.ds(base, CHUNK), :], acc_v)
        pltpu.async_copy(w_sh.at[idx_v], rows_v, sem).wait()

        def add_row(r, c2):
            for c in range(SLICES_PER_ROW):
                v = rows_v[r, pl.ds(c * LANES, LANES)]
                plsc.addupdate(acc_v.at[r, pl.ds(c * LANES, LANES)], v)
            return c2

        lax.fori_loop(0, CHUNK, add_row, 0, unroll=2)
        pltpu.sync_copy(acc_v, out_hbm.at[pl.ds(base, CHUNK), :])
        return carry

    lax.fori_loop(0, STEPS, step, 0)


@jax.jit
def _run(x, idx, w):
    mesh = plsc.VectorSubcoreMesh(core_axis_name="c", subcore_axis_name="s")
    f = pl.kernel(
        _body,
        out_type=jax.ShapeDtypeStruct((N, D), jnp.float32),
        mesh=mesh,
        scratch_types=[
            pltpu.VMEM_SHARED((NUM_EMB, 128), jnp.float32),
            pltpu.VMEM((CHUNK,), jnp.int32),
            pltpu.VMEM((CHUNK, D), jnp.float32),
            pltpu.VMEM((CHUNK, 128), jnp.float32),
            pltpu.SemaphoreType.DMA,
        ],
    )
    return f(x, idx, w)


def kernel(x, resids_positional_encoded, W):
    idx = resids_positional_encoded.astype(jnp.int32)
    w128 = jnp.pad(W, ((0, 0), (0, 128 - D)))
    return _run(x, idx, w128)


# speedup vs baseline: 3.4305x; 1.5054x over previous
import jax
import jax.numpy as jnp
from jax import lax
from jax.experimental import pallas as pl
from jax.experimental.pallas import tpu as pltpu
from jax.experimental.pallas import tpu_sc as plsc

NUM_EMB = 1000
D = 64
N = 819200

NC = 2
NS = 16
NW = NC * NS
LANES = 16

TOKENS_PER_WORKER = N // NW
CHUNK = 128
STEPS = TOKENS_PER_WORKER // CHUNK
SLICES_PER_ROW = D // LANES


def _body(x_hbm, idx_hbm, w_hbm, out_hbm, w_sh, idx_all,
          acc, rows, sem_g, sem_x, sem_out):
    cid = lax.axis_index("c")
    sid = lax.axis_index("s")
    wid = sid * NC + cid
    base0 = wid * TOKENS_PER_WORKER

    @pl.when(sid == 0)
    def _():
        pltpu.sync_copy(w_hbm, w_sh)

    pltpu.sync_copy(idx_hbm.at[pl.ds(base0, TOKENS_PER_WORKER)], idx_all)

    plsc.subcore_barrier()

    def idx_of(k):
        return idx_all.at[pl.ds(k * CHUNK, CHUNK)]

    def gather(k, b):
        return pltpu.make_async_copy(w_sh.at[idx_of(k)], rows[b], sem_g[b])

    def x_in(k, b):
        return pltpu.make_async_copy(
            x_hbm.at[pl.ds(base0 + k * CHUNK, CHUNK), :], acc[b], sem_x[b])

    def out_cp(k, b):
        return pltpu.make_async_copy(
            acc[b], out_hbm.at[pl.ds(base0 + k * CHUNK, CHUNK), :], sem_out[b])

    gather(0, 0).start()
    x_in(0, 0).start()

    def pair(g, carry):
        for b in (0, 1):
            k = 2 * g + b
            b1 = 1 - b
            @pl.when(k + 1 < STEPS)
            def _():
                gather(k + 1, b1).start()

            @pl.when((k + 1 < STEPS) & (k >= 1))
            def _():
                out_cp(k - 1, b1).wait()

            @pl.when(k + 1 < STEPS)
            def _():
                x_in(k + 1, b1).start()
            gather(k, b).wait()
            x_in(k, b).wait()

            @plsc.parallel_loop(0, CHUNK, 1, unroll=4)
            def add_row(r):
                for c in range(SLICES_PER_ROW):
                    v = rows[b][r, pl.ds(c * LANES, LANES)]
                    plsc.addupdate(acc[b].at[r, pl.ds(c * LANES, LANES)], v)

            out_cp(k, b).start()
        return carry

    lax.fori_loop(0, STEPS // 2, pair, 0)
    out_cp(STEPS - 2, 0).wait()
    out_cp(STEPS - 1, 1).wait()


@jax.jit
def _run(x, idx, w):
    mesh = plsc.VectorSubcoreMesh(core_axis_name="c", subcore_axis_name="s")
    f = pl.kernel(
        _body,
        out_type=jax.ShapeDtypeStruct((N, D), jnp.float32),
        mesh=mesh,
        scratch_types=[
            pltpu.VMEM_SHARED((NUM_EMB, 128), jnp.float32),
            pltpu.VMEM((TOKENS_PER_WORKER,), jnp.int32),
            [pltpu.VMEM((CHUNK, D), jnp.float32)] * 2,
            [pltpu.VMEM((CHUNK, 128), jnp.float32)] * 2,
            [pltpu.SemaphoreType.DMA] * 2,
            [pltpu.SemaphoreType.DMA] * 2,
            [pltpu.SemaphoreType.DMA] * 2,
        ],
    )
    return f(x, idx, w)


def kernel(x, resids_positional_encoded, W):
    idx = resids_positional_encoded.astype(jnp.int32)
    w128 = jnp.pad(W, ((0, 0), (0, 128 - D)))
    return _run(x, idx, w128)


# speedup vs baseline: 3.4319x; 1.0004x over previous
import jax
import jax.numpy as jnp
from jax import lax
from jax.experimental import pallas as pl
from jax.experimental.pallas import tpu as pltpu
from jax.experimental.pallas import tpu_sc as plsc

NUM_EMB = 1000
D = 64
N = 819200

NC = 2
NS = 16
NW = NC * NS
LANES = 16

TOKENS_PER_WORKER = N // NW
CHUNK = 128
STEPS = TOKENS_PER_WORKER // CHUNK
SLICES_PER_ROW = D // LANES


def _body(x_hbm, idx_hbm, w_hbm, out_hbm, w_sh, idx_all,
          acc, rows, sem_g, sem_x, sem_out):
    cid = lax.axis_index("c")
    sid = lax.axis_index("s")
    wid = sid * NC + cid
    base0 = wid * TOKENS_PER_WORKER

    @pl.when(sid == 0)
    def _():
        pltpu.sync_copy(w_hbm, w_sh)

    pltpu.sync_copy(idx_hbm.at[pl.ds(base0, TOKENS_PER_WORKER)], idx_all)

    plsc.subcore_barrier()

    def idx_of(k):
        return idx_all.at[pl.ds(k * CHUNK, CHUNK)]

    def gather(k, b):
        return pltpu.make_async_copy(w_sh.at[idx_of(k)], rows[b], sem_g[b])

    def x_in(k, b):
        return pltpu.make_async_copy(
            x_hbm.at[pl.ds(base0 + k * CHUNK, CHUNK), :], acc[b], sem_x[b])

    def out_cp(k, b):
        return pltpu.make_async_copy(
            acc[b], out_hbm.at[pl.ds(base0 + k * CHUNK, CHUNK), :], sem_out[b])

    gather(0, 0).start()
    x_in(0, 0).start()

    def pair(g, carry):
        for b in (0, 1):
            k = 2 * g + b
            b1 = 1 - b
            @pl.when(k + 1 < STEPS)
            def _():
                gather(k + 1, b1).start()

            @pl.when((k + 1 < STEPS) & (k >= 1))
            def _():
                out_cp(k - 1, b1).wait()

            @pl.when(k + 1 < STEPS)
            def _():
                x_in(k + 1, b1).start()
            gather(k, b).wait()
            x_in(k, b).wait()

            @plsc.parallel_loop(0, CHUNK, 1, unroll=4)
            def add_row(r):
                for c in range(SLICES_PER_ROW):
                    v = rows[b][r, pl.ds(c * LANES, LANES)]
                    plsc.addupdate(acc[b].at[r, pl.ds(c * LANES, LANES)], v)

            out_cp(k, b).start()
        return carry

    lax.fori_loop(0, STEPS // 2, pair, 0)
    out_cp(STEPS - 2, 0).wait()
    out_cp(STEPS - 1, 1).wait()


@jax.jit
def _run(x, idx, w):
    mesh = plsc.VectorSubcoreMesh(core_axis_name="c", subcore_axis_name="s")
    f = pl.kernel(
        _body,
        out_type=jax.ShapeDtypeStruct((N, D), jnp.float32),
        mesh=mesh,
        compiler_params=pltpu.CompilerParams(use_tc_tiling_on_sc=True),
        scratch_types=[
            pltpu.VMEM_SHARED((NUM_EMB, 128), jnp.float32),
            pltpu.VMEM((TOKENS_PER_WORKER,), jnp.int32),
            [pltpu.VMEM((CHUNK, D), jnp.float32)] * 2,
            [pltpu.VMEM((CHUNK, 128), jnp.float32)] * 2,
            [pltpu.SemaphoreType.DMA] * 2,
            [pltpu.SemaphoreType.DMA] * 2,
            [pltpu.SemaphoreType.DMA] * 2,
        ],
    )
    return f(x, idx, w)


def kernel(x, resids_positional_encoded, W):
    idx = resids_positional_encoded.astype(jnp.int32)
    w128 = jnp.pad(W, ((0, 0), (0, 128 - D)))
    return _run(x, idx, w128)
